# baseline (device time: 136307 ns/iter reference)
import jax
import jax.numpy as jnp
from jax import lax
from jax.experimental import pallas as pl
from jax.experimental.pallas import tpu as pltpu

N_DEV = 8
N_TOK = 2048
D = 1024
E_TOTAL = 64
E_LOCAL = E_TOTAL // N_DEV
CAP = 25
SLOTS = E_LOCAL * CAP


def _routing_table(route_idx):
    e = route_idx[:, 0]
    oh = (e[:, None] == jnp.arange(E_TOTAL, dtype=e.dtype)[None, :]).astype(
        jnp.int32
    )
    rank = jnp.sum(jnp.cumsum(oh, axis=0) * oh, axis=1) - 1
    keep = rank < CAP
    gslot = jnp.where(keep, e * CAP + rank, E_TOTAL * CAP)
    table = jnp.full((E_TOTAL * CAP + 1,), N_TOK, dtype=jnp.int32)
    table = table.at[gslot].set(jnp.arange(N_TOK, dtype=jnp.int32))
    return table[: E_TOTAL * CAP].reshape(N_DEV, SLOTS)


def _body(xg_ref, w_ref, out_ref, send_sems, recv_sems):
    my = lax.axis_index("i")
    left = lax.rem(my + N_DEV - 1, N_DEV)
    right = lax.rem(my + 1, N_DEV)

    barrier_sem = pltpu.get_barrier_semaphore()
    for nbr in (left, right):
        pl.semaphore_signal(
            barrier_sem,
            inc=1,
            device_id=(nbr,),
            device_id_type=pl.DeviceIdType.MESH,
        )
    pl.semaphore_wait(barrier_sem, 2)

    for k in range(E_LOCAL):
        blk = jnp.dot(
            xg_ref[k * CAP : (k + 1) * CAP, :],
            w_ref[k],
            preferred_element_type=jnp.float32,
        )
        out_ref[my, pl.ds(k * CAP, CAP), :] = blk

    for h in range(N_DEV - 1):
        slot = lax.rem(my - h + 2 * N_DEV, N_DEV)
        rdma = pltpu.make_async_remote_copy(
            src_ref=out_ref.at[slot],
            dst_ref=out_ref.at[slot],
            send_sem=send_sems.at[h],
            recv_sem=recv_sems.at[h],
            device_id=(right,),
            device_id_type=pl.DeviceIdType.MESH,
        )
        rdma.start()
        rdma.wait()


def kernel(x, router_W, route_idx, expert_W):
    del router_W
    table = _routing_table(route_idx)
    my = lax.axis_index("i")
    my_tok = table[my]
    xg = x[my_tok]

    gathered = pl.pallas_call(
        _body,
        out_shape=jax.ShapeDtypeStruct((N_DEV, SLOTS, D), jnp.float32),
        in_specs=[
            pl.BlockSpec(memory_space=pltpu.VMEM),
            pl.BlockSpec(memory_space=pltpu.VMEM),
        ],
        out_specs=pl.BlockSpec(memory_space=pltpu.VMEM),
        scratch_shapes=[
            pltpu.SemaphoreType.DMA((N_DEV - 1,)),
            pltpu.SemaphoreType.DMA((N_DEV - 1,)),
        ],
        compiler_params=pltpu.CompilerParams(collective_id=0),
    )(xg, expert_W)

    flat = gathered.reshape(N_DEV * SLOTS, D)
    out = jnp.zeros((N_TOK, D), jnp.float32)
    return out.at[table.reshape(-1)].set(flat, mode="drop")


# device time: 128880 ns/iter; 1.0576x vs baseline; 1.0576x over previous
import jax
import jax.numpy as jnp
from jax import lax
from jax.experimental import pallas as pl
from jax.experimental.pallas import tpu as pltpu

N_DEV = 8
N_TOK = 2048
D = 1024
E_TOTAL = 64
E_LOCAL = E_TOTAL // N_DEV
CAP = 25
SLOTS = E_LOCAL * CAP


def _routing_table(route_idx):
    e = route_idx[:, 0]
    oh = (e[:, None] == jnp.arange(E_TOTAL, dtype=e.dtype)[None, :]).astype(
        jnp.int32
    )
    rank = jnp.sum(jnp.cumsum(oh, axis=0) * oh, axis=1) - 1
    keep = rank < CAP
    gslot = jnp.where(keep, e * CAP + rank, E_TOTAL * CAP)
    table = jnp.full((E_TOTAL * CAP + 1,), N_TOK, dtype=jnp.int32)
    table = table.at[gslot].set(jnp.arange(N_TOK, dtype=jnp.int32))
    return table[: E_TOTAL * CAP].reshape(N_DEV, SLOTS)


def _body(xg_ref, w_ref, out_ref, send_sems, recv_sems):
    my = lax.axis_index("i")
    p4 = lax.rem(my, 4)
    plane4 = my - p4
    partners = (my ^ 1, plane4 + 3 - p4, my ^ 4)
    bases = (my, (my // 2) * 2, plane4)
    sizes = (1, 2, 4)

    barrier_sem = pltpu.get_barrier_semaphore()
    for nbr in partners:
        pl.semaphore_signal(
            barrier_sem,
            inc=1,
            device_id=(nbr,),
            device_id_type=pl.DeviceIdType.MESH,
        )
    pl.semaphore_wait(barrier_sem, 3)

    for k in range(E_LOCAL):
        blk = jnp.dot(
            xg_ref[k * CAP : (k + 1) * CAP, :],
            w_ref[k],
            preferred_element_type=jnp.float32,
        )
        out_ref[my, pl.ds(k * CAP, CAP), :] = blk

    for s in range(3):
        rdma = pltpu.make_async_remote_copy(
            src_ref=out_ref.at[pl.ds(bases[s], sizes[s])],
            dst_ref=out_ref.at[pl.ds(bases[s], sizes[s])],
            send_sem=send_sems.at[s],
            recv_sem=recv_sems.at[s],
            device_id=(partners[s],),
            device_id_type=pl.DeviceIdType.MESH,
        )
        rdma.start()
        rdma.wait()


def kernel(x, router_W, route_idx, expert_W):
    del router_W
    table = _routing_table(route_idx)
    my = lax.axis_index("i")
    my_tok = table[my]
    xg = x[my_tok]

    gathered = pl.pallas_call(
        _body,
        out_shape=jax.ShapeDtypeStruct((N_DEV, SLOTS, D), jnp.float32),
        in_specs=[
            pl.BlockSpec(memory_space=pltpu.VMEM),
            pl.BlockSpec(memory_space=pltpu.VMEM),
        ],
        out_specs=pl.BlockSpec(memory_space=pltpu.VMEM),
        scratch_shapes=[
            pltpu.SemaphoreType.DMA((3,)),
            pltpu.SemaphoreType.DMA((3,)),
        ],
        compiler_params=pltpu.CompilerParams(collective_id=0),
    )(xg, expert_W)

    flat = gathered.reshape(N_DEV * SLOTS, D)
    out = jnp.zeros((N_TOK, D), jnp.float32)
    return out.at[table.reshape(-1)].set(flat, mode="drop")
